# baseline (speedup 1.0000x reference)
import jax
import jax.numpy as jnp
from jax import lax
from jax.experimental import pallas as pl
from jax.experimental.pallas import tpu as pltpu
from jax.experimental.pallas import tpu_sc as plsc

_N = 10000
_H = 128
_NW = 32
_CHUNK = 64
_SUP = 16
_NSUP = 10
_NBUF = 5
_NCHUNK = _SUP * _NSUP
_EPW = _NCHUNK * _CHUNK
_ROWS_PER_TILE = 640
_NPAD = 16 * _ROWS_PER_TILE
_NZC = _ROWS_PER_TILE // _CHUNK


def _make_agg():
  mesh = plsc.VectorSubcoreMesh(core_axis_name="c", subcore_axis_name="s")
  out_type = jax.ShapeDtypeStruct((2, _NPAD, _H), jnp.float32)
  scratch = [
      pltpu.VMEM((_SUP, _CHUNK), jnp.int32),
      pltpu.VMEM((_SUP, _CHUNK), jnp.int32),
      pltpu.VMEM((_NBUF, _CHUNK, _H), jnp.float32),
      pltpu.VMEM_SHARED((_NPAD, _H), jnp.float32),
      [pltpu.SemaphoreType.DMA] * _NBUF,
      [pltpu.SemaphoreType.DMA] * _NBUF,
  ]

  def body(h_hbm, src_hbm, dst_hbm, z_hbm, agg_out, src_v, dst_v, rows_v,
           acc_s, gsems, ssems):
    c = lax.axis_index("c")
    s = lax.axis_index("s")
    w = c * 16 + s
    base = s * _ROWS_PER_TILE
    rows = [rows_v.at[jnp.int32(b)] for b in range(_NBUF)]

    pltpu.sync_copy(z_hbm, rows[0])
    for k in range(_NZC):
      pltpu.sync_copy(rows[0], acc_s.at[pl.ds(base + k * _CHUNK, _CHUNK)])
    plsc.subcore_barrier()

    la = _NBUF - 1

    def step(g, carry):
      off = g * _SUP
      pltpu.sync_copy(src_hbm.at[w].at[pl.ds(off, _SUP)], src_v)
      pltpu.sync_copy(dst_hbm.at[w].at[pl.ds(off, _SUP)], dst_v)
      descs_g = [None] * _NBUF
      descs_s = [None] * _NBUF
      for k in range(la):
        descs_g[k] = pltpu.async_copy(
            h_hbm.at[src_v.at[jnp.int32(k)]], rows[k], gsems[k])
      for j in range(_SUP):
        b = j % _NBUF
        if j + la < _SUP:
          bg = (j + la) % _NBUF
          if descs_s[bg] is not None:
            descs_s[bg].wait()
          descs_g[bg] = pltpu.async_copy(
              h_hbm.at[src_v.at[jnp.int32(j + la)]], rows[bg], gsems[bg])
        descs_g[b].wait()
        descs_s[b] = pltpu.async_copy(
            rows[b], acc_s.at[dst_v.at[jnp.int32(j)]], ssems[b], add=True)
      for b in range(_NBUF):
        if descs_s[b] is not None:
          descs_s[b].wait()
      return carry

    lax.fori_loop(jnp.int32(0), jnp.int32(_NSUP), step, 0)
    plsc.subcore_barrier()

    sl = pl.ds(base, _ROWS_PER_TILE)
    pltpu.sync_copy(acc_s.at[sl], agg_out.at[c].at[sl])

  return pl.kernel(body, out_type=out_type, mesh=mesh, scratch_types=scratch)


def _make_cnt():
  mesh = plsc.VectorSubcoreMesh(core_axis_name="c", subcore_axis_name="s")
  out_type = jax.ShapeDtypeStruct((2, _NPAD, _H), jnp.float32)
  scratch = [
      pltpu.VMEM((_SUP, _CHUNK), jnp.int32),
      pltpu.VMEM((_CHUNK, _H), jnp.float32),
      pltpu.VMEM_SHARED((_NPAD, _H), jnp.float32),
  ]

  def body(dst_hbm, z_hbm, ones_hbm, cnt_out, dst_v, ones_v, acc_s):
    c = lax.axis_index("c")
    s = lax.axis_index("s")
    w = c * 16 + s
    base = s * _ROWS_PER_TILE

    pltpu.sync_copy(z_hbm, ones_v)
    for k in range(_NZC):
      pltpu.sync_copy(ones_v, acc_s.at[pl.ds(base + k * _CHUNK, _CHUNK)])
    pltpu.sync_copy(ones_hbm, ones_v)
    plsc.subcore_barrier()

    def step(g, carry):
      off = g * _SUP
      pltpu.sync_copy(dst_hbm.at[w].at[pl.ds(off, _SUP)], dst_v)
      for j in range(_SUP):
        pltpu.sync_copy(ones_v, acc_s.at[dst_v.at[jnp.int32(j)]], add=True)
      return carry

    lax.fori_loop(jnp.int32(0), jnp.int32(_NSUP), step, 0)
    plsc.subcore_barrier()

    sl = pl.ds(base, _ROWS_PER_TILE)
    pltpu.sync_copy(acc_s.at[sl], cnt_out.at[c].at[sl])

  return pl.kernel(body, out_type=out_type, mesh=mesh, scratch_types=scratch)


def _in_proj(x_ref, w_ref, b_ref, o_ref):
  o_ref[...] = jnp.maximum(
      jnp.dot(x_ref[...], w_ref[...], preferred_element_type=jnp.float32)
      + b_ref[...], 0.0)


def _lin_r(h_ref, wr_ref, o_ref):
  o_ref[...] = jnp.dot(h_ref[...], wr_ref[...],
                       preferred_element_type=jnp.float32)


def _make_layer(residual, first):
  def body(aggp, cnt_in, h_ref, hr_ref, wl_ref, bl_ref, g_ref, bt_ref, *outs):
    agg = aggp[0, :_N, :] + aggp[1, :_N, :]
    if first:
      o_ref, cnt_out = outs
      cnt = jnp.maximum(cnt_in[0, :_N, 0:1] + cnt_in[1, :_N, 0:1], 1.0)
      cnt_out[...] = jnp.broadcast_to(cnt, (_N, 128))
    else:
      (o_ref,) = outs
      cnt = cnt_in[:, 0:1]
    mean = agg / cnt
    h = h_ref[...]
    z = (jnp.dot(mean, wl_ref[...], preferred_element_type=jnp.float32)
         + bl_ref[...] + hr_ref[...])
    mu = jnp.mean(z, axis=0, keepdims=True)
    zc = z - mu
    var = jnp.mean(zc * zc, axis=0, keepdims=True)
    a = jnp.maximum(g_ref[...] * zc * lax.rsqrt(var + 1e-5) + bt_ref[...], 0.0)
    o_ref[...] = a + h if residual else a
  return body


def _head(h_ref, w1_ref, b1_ref, w2_ref, b2_ref, o_ref):
  t = jnp.maximum(
      jnp.dot(h_ref[...], w1_ref[...], preferred_element_type=jnp.float32)
      + b1_ref[...], 0.0)
  o_ref[...] = (jnp.dot(t, w2_ref[...], preferred_element_type=jnp.float32)
                + b2_ref[...])


def kernel(x, edge_index, params):
  x = x.astype(jnp.float32)
  ei = edge_index.astype(jnp.int32)
  srcw = jnp.pad(ei[0].reshape(_NW, _N), ((0, 0), (0, _EPW - _N)),
                 constant_values=0).reshape(_NW, _NCHUNK, _CHUNK)
  dstw = jnp.pad(ei[1].reshape(_NW, _N), ((0, 0), (0, _EPW - _N)),
                 constant_values=_N).reshape(_NW, _NCHUNK, _CHUNK)
  zH = jnp.zeros((_CHUNK, _H), jnp.float32)
  onesH = jnp.ones((_CHUNK, _H), jnp.float32)

  agg = _make_agg()
  cntk = _make_cnt()

  p = params
  h = pl.pallas_call(
      _in_proj, out_shape=jax.ShapeDtypeStruct((_N, _H), jnp.float32),
  )(x, p['in_w'].T, p['in_b'][None, :])

  cntp = cntk(dstw, zH, onesH)

  cnt_in = cntp
  for i in range(3):
    cp = p['convs'][i]
    bp = p['bns'][i]
    aggp = agg(h, srcw, dstw, zH)
    hr = pl.pallas_call(
        _lin_r, out_shape=jax.ShapeDtypeStruct((_N, _H), jnp.float32),
    )(h, cp['lin_r_w'].T)
    if i == 0:
      outs = [jax.ShapeDtypeStruct((_N, _H), jnp.float32),
              jax.ShapeDtypeStruct((_N, _H), jnp.float32)]
      h, cnt_in = pl.pallas_call(_make_layer(False, True), out_shape=outs)(
          aggp, cnt_in, h, hr, cp['lin_l_w'].T, cp['lin_l_b'][None, :],
          bp['gamma'][None, :], bp['beta'][None, :])
    else:
      h = pl.pallas_call(
          _make_layer(True, False),
          out_shape=jax.ShapeDtypeStruct((_N, _H), jnp.float32))(
          aggp, cnt_in, h, hr, cp['lin_l_w'].T, cp['lin_l_b'][None, :],
          bp['gamma'][None, :], bp['beta'][None, :])

  w1 = jnp.zeros((_H, _H), jnp.float32).at[:, :64].set(p['fc1_w'].T)
  b1 = jnp.zeros((1, _H), jnp.float32).at[0, :64].set(p['fc1_b'])
  w2 = jnp.zeros((_H, _H), jnp.float32).at[:64, :2].set(p['fc2_w'].T)
  b2 = jnp.zeros((1, _H), jnp.float32).at[0, :2].set(p['fc2_b'])
  out = pl.pallas_call(
      _head, out_shape=jax.ShapeDtypeStruct((_N, _H), jnp.float32),
  )(h, w1, b1, w2, b2)
  return out[:, :2], h


# speedup vs baseline: 1.0083x; 1.0083x over previous
import jax
import jax.numpy as jnp
from jax import lax
from jax.experimental import pallas as pl
from jax.experimental.pallas import tpu as pltpu
from jax.experimental.pallas import tpu_sc as plsc

_N = 10000
_H = 128
_NW = 32
_CHUNK = 64
_SUP = 16
_NSUP = 10
_NBUF = 4
_NCHUNK = _SUP * _NSUP
_EPW = _NCHUNK * _CHUNK
_ROWS_PER_TILE = 640
_NPAD = 16 * _ROWS_PER_TILE
_NZC = _ROWS_PER_TILE // _CHUNK


def _make_agg():
  mesh = plsc.VectorSubcoreMesh(core_axis_name="c", subcore_axis_name="s")
  out_type = jax.ShapeDtypeStruct((2, _NPAD, _H), jnp.float32)
  scratch = [
      pltpu.VMEM((_SUP, _CHUNK), jnp.int32),
      pltpu.VMEM((_SUP, _CHUNK), jnp.int32),
      pltpu.VMEM((_NBUF, _CHUNK, _H), jnp.float32),
      pltpu.VMEM_SHARED((_NPAD, _H), jnp.float32),
      [pltpu.SemaphoreType.DMA] * _NBUF,
      [pltpu.SemaphoreType.DMA] * _NBUF,
  ]

  def body(h_hbm, src_hbm, dst_hbm, z_hbm, agg_out, src_v, dst_v, rows_v,
           acc_s, gsems, ssems):
    c = lax.axis_index("c")
    s = lax.axis_index("s")
    w = c * 16 + s
    base = s * _ROWS_PER_TILE
    rows = [rows_v.at[jnp.int32(b)] for b in range(_NBUF)]

    pltpu.sync_copy(z_hbm, rows[0])
    for k in range(_NZC):
      pltpu.sync_copy(rows[0], acc_s.at[pl.ds(base + k * _CHUNK, _CHUNK)])
    plsc.subcore_barrier()

    la = _NBUF - 1

    def step(g, carry):
      off = g * _SUP
      pltpu.sync_copy(src_hbm.at[w].at[pl.ds(off, _SUP)], src_v)
      pltpu.sync_copy(dst_hbm.at[w].at[pl.ds(off, _SUP)], dst_v)
      descs_g = [None] * _NBUF
      descs_s = [None] * _NBUF
      for k in range(la):
        descs_g[k] = pltpu.async_copy(
            h_hbm.at[src_v.at[jnp.int32(k)]], rows[k], gsems[k])
      for j in range(_SUP):
        b = j % _NBUF
        if j + la < _SUP:
          bg = (j + la) % _NBUF
          if descs_s[bg] is not None:
            descs_s[bg].wait()
          descs_g[bg] = pltpu.async_copy(
              h_hbm.at[src_v.at[jnp.int32(j + la)]], rows[bg], gsems[bg])
        descs_g[b].wait()
        descs_s[b] = pltpu.async_copy(
            rows[b], acc_s.at[dst_v.at[jnp.int32(j)]], ssems[b], add=True)
      for b in range(_NBUF):
        if descs_s[b] is not None:
          descs_s[b].wait()
      return carry

    lax.fori_loop(jnp.int32(0), jnp.int32(_NSUP), step, 0)
    plsc.subcore_barrier()

    sl = pl.ds(base, _ROWS_PER_TILE)
    pltpu.sync_copy(acc_s.at[sl], agg_out.at[c].at[sl])

  return pl.kernel(body, out_type=out_type, mesh=mesh, scratch_types=scratch)


def _make_cnt():
  mesh = plsc.VectorSubcoreMesh(core_axis_name="c", subcore_axis_name="s")
  out_type = jax.ShapeDtypeStruct((2, _NPAD, _H), jnp.float32)
  scratch = [
      pltpu.VMEM((_SUP, _CHUNK), jnp.int32),
      pltpu.VMEM((_CHUNK, _H), jnp.float32),
      pltpu.VMEM_SHARED((_NPAD, _H), jnp.float32),
  ]

  def body(dst_hbm, z_hbm, ones_hbm, cnt_out, dst_v, ones_v, acc_s):
    c = lax.axis_index("c")
    s = lax.axis_index("s")
    w = c * 16 + s
    base = s * _ROWS_PER_TILE

    pltpu.sync_copy(z_hbm, ones_v)
    for k in range(_NZC):
      pltpu.sync_copy(ones_v, acc_s.at[pl.ds(base + k * _CHUNK, _CHUNK)])
    pltpu.sync_copy(ones_hbm, ones_v)
    plsc.subcore_barrier()

    def step(g, carry):
      off = g * _SUP
      pltpu.sync_copy(dst_hbm.at[w].at[pl.ds(off, _SUP)], dst_v)
      for j in range(_SUP):
        pltpu.sync_copy(ones_v, acc_s.at[dst_v.at[jnp.int32(j)]], add=True)
      return carry

    lax.fori_loop(jnp.int32(0), jnp.int32(_NSUP), step, 0)
    plsc.subcore_barrier()

    sl = pl.ds(base, _ROWS_PER_TILE)
    pltpu.sync_copy(acc_s.at[sl], cnt_out.at[c].at[sl])

  return pl.kernel(body, out_type=out_type, mesh=mesh, scratch_types=scratch)


def _in_proj(x_ref, w_ref, b_ref, o_ref):
  o_ref[...] = jnp.maximum(
      jnp.dot(x_ref[...], w_ref[...], preferred_element_type=jnp.float32)
      + b_ref[...], 0.0)


def _make_layer(residual, first):
  def body(aggp, cnt_in, h_ref, wl_ref, bl_ref, wr_ref, g_ref, bt_ref, *outs):
    agg = aggp[0, :_N, :] + aggp[1, :_N, :]
    if first:
      o_ref, cnt_out = outs
      cnt = jnp.maximum(cnt_in[0, :_N, 0:1] + cnt_in[1, :_N, 0:1], 1.0)
      cnt_out[...] = jnp.broadcast_to(cnt, (_N, 128))
    else:
      (o_ref,) = outs
      cnt = cnt_in[:, 0:1]
    mean = agg / cnt
    h = h_ref[...]
    z = (jnp.dot(mean, wl_ref[...], preferred_element_type=jnp.float32)
         + bl_ref[...]
         + jnp.dot(h, wr_ref[...], preferred_element_type=jnp.float32))
    mu = jnp.mean(z, axis=0, keepdims=True)
    zc = z - mu
    var = jnp.mean(zc * zc, axis=0, keepdims=True)
    a = jnp.maximum(g_ref[...] * zc * lax.rsqrt(var + 1e-5) + bt_ref[...], 0.0)
    o_ref[...] = a + h if residual else a
  return body


def _head(h_ref, w1_ref, b1_ref, w2_ref, b2_ref, o_ref):
  t = jnp.maximum(
      jnp.dot(h_ref[...], w1_ref[...], preferred_element_type=jnp.float32)
      + b1_ref[...], 0.0)
  o_ref[...] = (jnp.dot(t, w2_ref[...], preferred_element_type=jnp.float32)
                + b2_ref[...])


def kernel(x, edge_index, params):
  x = x.astype(jnp.float32)
  ei = edge_index.astype(jnp.int32)
  srcw = jnp.pad(ei[0].reshape(_NW, _N), ((0, 0), (0, _EPW - _N)),
                 constant_values=0).reshape(_NW, _NCHUNK, _CHUNK)
  dstw = jnp.pad(ei[1].reshape(_NW, _N), ((0, 0), (0, _EPW - _N)),
                 constant_values=_N).reshape(_NW, _NCHUNK, _CHUNK)
  zH = jnp.zeros((_CHUNK, _H), jnp.float32)
  onesH = jnp.ones((_CHUNK, _H), jnp.float32)

  agg = _make_agg()
  cntk = _make_cnt()

  p = params
  h = pl.pallas_call(
      _in_proj, out_shape=jax.ShapeDtypeStruct((_N, _H), jnp.float32),
  )(x, p['in_w'].T, p['in_b'][None, :])

  cntp = cntk(dstw, zH, onesH)

  cnt_in = cntp
  for i in range(3):
    cp = p['convs'][i]
    bp = p['bns'][i]
    aggp = agg(h, srcw, dstw, zH)
    if i == 0:
      outs = [jax.ShapeDtypeStruct((_N, _H), jnp.float32),
              jax.ShapeDtypeStruct((_N, _H), jnp.float32)]
      h, cnt_in = pl.pallas_call(_make_layer(False, True), out_shape=outs)(
          aggp, cnt_in, h, cp['lin_l_w'].T, cp['lin_l_b'][None, :],
          cp['lin_r_w'].T, bp['gamma'][None, :], bp['beta'][None, :])
    else:
      h = pl.pallas_call(
          _make_layer(True, False),
          out_shape=jax.ShapeDtypeStruct((_N, _H), jnp.float32))(
          aggp, cnt_in, h, cp['lin_l_w'].T, cp['lin_l_b'][None, :],
          cp['lin_r_w'].T, bp['gamma'][None, :], bp['beta'][None, :])

  w1 = jnp.zeros((_H, _H), jnp.float32).at[:, :64].set(p['fc1_w'].T)
  b1 = jnp.zeros((1, _H), jnp.float32).at[0, :64].set(p['fc1_b'])
  w2 = jnp.zeros((_H, _H), jnp.float32).at[:64, :2].set(p['fc2_w'].T)
  b2 = jnp.zeros((1, _H), jnp.float32).at[0, :2].set(p['fc2_b'])
  out = pl.pallas_call(
      _head, out_shape=jax.ShapeDtypeStruct((_N, _H), jnp.float32),
  )(h, w1, b1, w2, b2)
  return out[:, :2], h
